# baseline (device time: 80807 ns/iter reference)
import jax
import jax.numpy as jnp
from jax import lax
from jax.experimental import pallas as pl
from jax.experimental.pallas import tpu as pltpu

N_DEV = 4


def kernel(x, router_W, route_idx, expert_W):
    n_tok, d = x.shape
    n_exp = router_W.shape[1]
    e_loc, _, h = expert_W.shape

    def body(x_ref, rw_ref, idx_ref, ew_ref, out_ref, comm_ref, send_sems, recv_sems):
        my = lax.axis_index("i")
        left = lax.rem(my + N_DEV - 1, N_DEV)
        right = lax.rem(my + 1, N_DEV)

        barrier_sem = pltpu.get_barrier_semaphore()
        for nbr in [left, right]:
            pl.semaphore_signal(
                barrier_sem, inc=1,
                device_id=(nbr,), device_id_type=pl.DeviceIdType.MESH,
            )
        pl.semaphore_wait(barrier_sem, 2)

        xv = x_ref[:, :]
        scores = jnp.dot(xv, rw_ref[:, :], preferred_element_type=jnp.float32)
        s_max = jnp.max(scores, axis=-1, keepdims=True)
        p = jnp.exp(scores - s_max)
        p = p / jnp.sum(p, axis=-1, keepdims=True)
        col = lax.broadcasted_iota(jnp.int32, (n_tok, n_exp), 1)
        chosen = jnp.logical_or(col == idx_ref[:, 0:1], col == idx_ref[:, 1:2])
        g = jnp.where(chosen, p, 0.0)
        gates = g / jnp.sum(g, axis=-1, keepdims=True)

        def chunk_contrib(src_ref, origin):
            part = jnp.zeros((n_tok, h), jnp.float32)
            for j in range(e_loc):
                e_id = origin * e_loc + j
                w = jnp.sum(jnp.where(col == e_id, gates, 0.0), axis=-1)
                y = jnp.dot(xv, src_ref[j, :, :], preferred_element_type=jnp.float32)
                part = part + w[:, None] * y
            return part

        for hp in range(N_DEV - 1):
            src = ew_ref if hp == 0 else comm_ref.at[hp - 1]
            rdma = pltpu.make_async_remote_copy(
                src_ref=src,
                dst_ref=comm_ref.at[hp],
                send_sem=send_sems.at[hp],
                recv_sem=recv_sems.at[hp],
                device_id=(right,),
                device_id_type=pl.DeviceIdType.MESH,
            )
            rdma.start()
            origin = lax.rem(my - hp + N_DEV, N_DEV)
            part = chunk_contrib(src, origin)
            if hp == 0:
                out_ref[:, :] = part
            else:
                out_ref[:, :] += part
            rdma.wait()

        out_ref[:, :] += chunk_contrib(
            comm_ref.at[N_DEV - 2], lax.rem(my + 1, N_DEV)
        )

    return pl.pallas_call(
        body,
        out_shape=jax.ShapeDtypeStruct((n_tok, h), jnp.float32),
        in_specs=[
            pl.BlockSpec(memory_space=pltpu.VMEM),
            pl.BlockSpec(memory_space=pltpu.VMEM),
            pl.BlockSpec(memory_space=pltpu.VMEM),
            pl.BlockSpec(memory_space=pltpu.VMEM),
        ],
        out_specs=pl.BlockSpec(memory_space=pltpu.VMEM),
        scratch_shapes=[
            pltpu.VMEM((N_DEV - 1, e_loc, d, h), jnp.float32),
            pltpu.SemaphoreType.DMA((N_DEV - 1,)),
            pltpu.SemaphoreType.DMA((N_DEV - 1,)),
        ],
        compiler_params=pltpu.CompilerParams(collective_id=0),
    )(x, router_W, route_idx, expert_W)


# device time: 46180 ns/iter; 1.7498x vs baseline; 1.7498x over previous
import jax
import jax.numpy as jnp
from jax import lax
from jax.experimental import pallas as pl
from jax.experimental.pallas import tpu as pltpu

N_DEV = 4


def kernel(x, router_W, route_idx, expert_W):
    n_tok, d = x.shape
    n_exp = router_W.shape[1]
    e_loc, _, h = expert_W.shape
    e_half = e_loc // 2

    def body(x_ref, rw_ref, idx_ref, ew_ref, out_ref,
             slotL, slotR, slotO, sendA, recvA, sendB, recvB):
        my = lax.axis_index("i")
        left = lax.rem(my + N_DEV - 1, N_DEV)
        right = lax.rem(my + 1, N_DEV)

        barrier_sem = pltpu.get_barrier_semaphore()
        for nbr in [left, right]:
            pl.semaphore_signal(
                barrier_sem, inc=1,
                device_id=(nbr,), device_id_type=pl.DeviceIdType.MESH,
            )
        pl.semaphore_wait(barrier_sem, 2)

        a_right = pltpu.make_async_remote_copy(
            src_ref=ew_ref, dst_ref=slotL,
            send_sem=sendA.at[0], recv_sem=recvA.at[0],
            device_id=(right,), device_id_type=pl.DeviceIdType.MESH,
        )
        a_left = pltpu.make_async_remote_copy(
            src_ref=ew_ref, dst_ref=slotR,
            send_sem=sendA.at[1], recv_sem=recvA.at[1],
            device_id=(left,), device_id_type=pl.DeviceIdType.MESH,
        )
        a_right.start()
        a_left.start()

        xv = x_ref[:, :]
        scores = jnp.dot(xv, rw_ref[:, :], preferred_element_type=jnp.float32)
        s_max = jnp.max(scores, axis=-1, keepdims=True)
        p = jnp.exp(scores - s_max)
        p = p / jnp.sum(p, axis=-1, keepdims=True)
        col = lax.broadcasted_iota(jnp.int32, (n_tok, n_exp), 1)
        chosen = jnp.logical_or(col == idx_ref[:, 0:1], col == idx_ref[:, 1:2])
        g = jnp.where(chosen, p, 0.0)
        gates = g / jnp.sum(g, axis=-1, keepdims=True)

        def chunk_contrib(src_ref, origin):
            part = jnp.zeros((n_tok, h), jnp.float32)
            for j in range(e_loc):
                e_id = origin * e_loc + j
                w = jnp.sum(jnp.where(col == e_id, gates, 0.0), axis=-1)
                y = jnp.dot(xv, src_ref[j, :, :], preferred_element_type=jnp.float32)
                part = part + w[:, None] * y
            return part

        out_ref[:, :] = chunk_contrib(ew_ref, my)

        a_right.wait_recv()
        a_left.wait_recv()

        b_right = pltpu.make_async_remote_copy(
            src_ref=slotL.at[pl.ds(e_half, e_half)],
            dst_ref=slotO.at[pl.ds(e_half, e_half)],
            send_sem=sendB.at[0], recv_sem=recvB.at[0],
            device_id=(right,), device_id_type=pl.DeviceIdType.MESH,
        )
        b_left = pltpu.make_async_remote_copy(
            src_ref=slotR.at[pl.ds(0, e_half)],
            dst_ref=slotO.at[pl.ds(0, e_half)],
            send_sem=sendB.at[1], recv_sem=recvB.at[1],
            device_id=(left,), device_id_type=pl.DeviceIdType.MESH,
        )
        b_right.start()
        b_left.start()

        out_ref[:, :] += chunk_contrib(slotL, left)
        out_ref[:, :] += chunk_contrib(slotR, right)

        b_right.wait_recv()
        b_left.wait_recv()

        out_ref[:, :] += chunk_contrib(slotO, lax.rem(my + 2, N_DEV))

        a_right.wait_send()
        a_left.wait_send()
        b_right.wait_send()
        b_left.wait_send()

    return pl.pallas_call(
        body,
        out_shape=jax.ShapeDtypeStruct((n_tok, h), jnp.float32),
        in_specs=[
            pl.BlockSpec(memory_space=pltpu.VMEM),
            pl.BlockSpec(memory_space=pltpu.VMEM),
            pl.BlockSpec(memory_space=pltpu.VMEM),
            pl.BlockSpec(memory_space=pltpu.VMEM),
        ],
        out_specs=pl.BlockSpec(memory_space=pltpu.VMEM),
        scratch_shapes=[
            pltpu.VMEM((e_loc, d, h), jnp.float32),
            pltpu.VMEM((e_loc, d, h), jnp.float32),
            pltpu.VMEM((e_loc, d, h), jnp.float32),
            pltpu.SemaphoreType.DMA((2,)),
            pltpu.SemaphoreType.DMA((2,)),
            pltpu.SemaphoreType.DMA((2,)),
            pltpu.SemaphoreType.DMA((2,)),
        ],
        compiler_params=pltpu.CompilerParams(collective_id=0),
    )(x, router_W, route_idx, expert_W)


# device time: 45198 ns/iter; 1.7878x vs baseline; 1.0217x over previous
import jax
import jax.numpy as jnp
from jax import lax
from jax.experimental import pallas as pl
from jax.experimental.pallas import tpu as pltpu

N_DEV = 4


def kernel(x, router_W, route_idx, expert_W):
    n_tok, d = x.shape
    n_exp = router_W.shape[1]
    e_loc, _, h = expert_W.shape
    e_half = e_loc // 2

    def body(x_ref, rw_ref, idx_ref, ew_ref, out_ref,
             slotL, slotR, slotO, sendA, recvA, sendB, recvB):
        my = lax.axis_index("i")
        left = lax.rem(my + N_DEV - 1, N_DEV)
        right = lax.rem(my + 1, N_DEV)

        barrier_sem = pltpu.get_barrier_semaphore()
        for nbr in [left, right]:
            pl.semaphore_signal(
                barrier_sem, inc=1,
                device_id=(nbr,), device_id_type=pl.DeviceIdType.MESH,
            )
        pl.semaphore_wait(barrier_sem, 2)

        lo = pl.ds(0, e_half)
        hi = pl.ds(e_half, e_half)
        a_r1 = pltpu.make_async_remote_copy(
            src_ref=ew_ref.at[hi], dst_ref=slotL.at[hi],
            send_sem=sendA.at[0], recv_sem=recvA.at[0],
            device_id=(right,), device_id_type=pl.DeviceIdType.MESH,
        )
        a_r0 = pltpu.make_async_remote_copy(
            src_ref=ew_ref.at[lo], dst_ref=slotL.at[lo],
            send_sem=sendA.at[1], recv_sem=recvA.at[1],
            device_id=(right,), device_id_type=pl.DeviceIdType.MESH,
        )
        a_l0 = pltpu.make_async_remote_copy(
            src_ref=ew_ref.at[lo], dst_ref=slotR.at[lo],
            send_sem=sendA.at[2], recv_sem=recvA.at[2],
            device_id=(left,), device_id_type=pl.DeviceIdType.MESH,
        )
        a_l1 = pltpu.make_async_remote_copy(
            src_ref=ew_ref.at[hi], dst_ref=slotR.at[hi],
            send_sem=sendA.at[3], recv_sem=recvA.at[3],
            device_id=(left,), device_id_type=pl.DeviceIdType.MESH,
        )
        a_r1.start()
        a_r0.start()
        a_l0.start()
        a_l1.start()

        xv = x_ref[:, :]
        scores = jnp.dot(xv, rw_ref[:, :], preferred_element_type=jnp.float32)
        s_max = jnp.max(scores, axis=-1, keepdims=True)
        p = jnp.exp(scores - s_max)
        p = p / jnp.sum(p, axis=-1, keepdims=True)
        col = lax.broadcasted_iota(jnp.int32, (n_tok, n_exp), 1)
        chosen = jnp.logical_or(col == idx_ref[:, 0:1], col == idx_ref[:, 1:2])
        g = jnp.where(chosen, p, 0.0)
        gates = g / jnp.sum(g, axis=-1, keepdims=True)

        def contrib(src_ref, origin, j0, j1):
            part = jnp.zeros((n_tok, h), jnp.float32)
            for j in range(j0, j1):
                e_id = origin * e_loc + j
                w = jnp.sum(jnp.where(col == e_id, gates, 0.0), axis=-1)
                y = jnp.dot(xv, src_ref[j, :, :], preferred_element_type=jnp.float32)
                part = part + w[:, None] * y
            return part

        out_ref[:, :] = contrib(ew_ref, my, 0, e_loc)

        a_r1.wait_recv()
        b_right = pltpu.make_async_remote_copy(
            src_ref=slotL.at[hi], dst_ref=slotO.at[hi],
            send_sem=sendB.at[0], recv_sem=recvB.at[0],
            device_id=(right,), device_id_type=pl.DeviceIdType.MESH,
        )
        b_right.start()
        a_l0.wait_recv()
        b_left = pltpu.make_async_remote_copy(
            src_ref=slotR.at[lo], dst_ref=slotO.at[lo],
            send_sem=sendB.at[1], recv_sem=recvB.at[1],
            device_id=(left,), device_id_type=pl.DeviceIdType.MESH,
        )
        b_left.start()

        out_ref[:, :] += contrib(slotL, left, e_half, e_loc)
        out_ref[:, :] += contrib(slotR, right, 0, e_half)

        a_r0.wait_recv()
        out_ref[:, :] += contrib(slotL, left, 0, e_half)
        a_l1.wait_recv()
        out_ref[:, :] += contrib(slotR, right, e_half, e_loc)

        opp = lax.rem(my + 2, N_DEV)
        b_right.wait_recv()
        out_ref[:, :] += contrib(slotO, opp, e_half, e_loc)
        b_left.wait_recv()
        out_ref[:, :] += contrib(slotO, opp, 0, e_half)

        for flow in (a_r1, a_r0, a_l0, a_l1, b_right, b_left):
            flow.wait_send()

    return pl.pallas_call(
        body,
        out_shape=jax.ShapeDtypeStruct((n_tok, h), jnp.float32),
        in_specs=[
            pl.BlockSpec(memory_space=pltpu.VMEM),
            pl.BlockSpec(memory_space=pltpu.VMEM),
            pl.BlockSpec(memory_space=pltpu.VMEM),
            pl.BlockSpec(memory_space=pltpu.VMEM),
        ],
        out_specs=pl.BlockSpec(memory_space=pltpu.VMEM),
        scratch_shapes=[
            pltpu.VMEM((e_loc, d, h), jnp.float32),
            pltpu.VMEM((e_loc, d, h), jnp.float32),
            pltpu.VMEM((e_loc, d, h), jnp.float32),
            pltpu.SemaphoreType.DMA((4,)),
            pltpu.SemaphoreType.DMA((4,)),
            pltpu.SemaphoreType.DMA((2,)),
            pltpu.SemaphoreType.DMA((2,)),
        ],
        compiler_params=pltpu.CompilerParams(collective_id=0),
    )(x, router_W, route_idx, expert_W)


# device time: 44996 ns/iter; 1.7959x vs baseline; 1.0045x over previous
import jax
import jax.numpy as jnp
from jax import lax
from jax.experimental import pallas as pl
from jax.experimental.pallas import tpu as pltpu

N_DEV = 4


def kernel(x, router_W, route_idx, expert_W):
    n_tok, d = x.shape
    n_exp = router_W.shape[1]
    e_loc, _, h = expert_W.shape
    e_half = e_loc // 2

    def body(x_ref, rw_ref, idx_ref, ew_ref, out_ref,
             slotL, slotR, slotO, sendA, recvA, sendB, recvB):
        my = lax.axis_index("i")
        left = lax.rem(my + N_DEV - 1, N_DEV)
        right = lax.rem(my + 1, N_DEV)

        barrier_sem = pltpu.get_barrier_semaphore()
        for nbr in [left, right]:
            pl.semaphore_signal(
                barrier_sem, inc=1,
                device_id=(nbr,), device_id_type=pl.DeviceIdType.MESH,
            )
        pl.semaphore_wait(barrier_sem, 2)

        lo = pl.ds(0, e_half)
        hi = pl.ds(e_half, e_half)
        a_r1 = pltpu.make_async_remote_copy(
            src_ref=ew_ref.at[hi], dst_ref=slotL.at[hi],
            send_sem=sendA.at[0], recv_sem=recvA.at[0],
            device_id=(right,), device_id_type=pl.DeviceIdType.MESH,
        )
        a_r0 = pltpu.make_async_remote_copy(
            src_ref=ew_ref.at[lo], dst_ref=slotL.at[lo],
            send_sem=sendA.at[1], recv_sem=recvA.at[1],
            device_id=(right,), device_id_type=pl.DeviceIdType.MESH,
        )
        a_l0 = pltpu.make_async_remote_copy(
            src_ref=ew_ref.at[lo], dst_ref=slotR.at[lo],
            send_sem=sendA.at[2], recv_sem=recvA.at[2],
            device_id=(left,), device_id_type=pl.DeviceIdType.MESH,
        )
        a_l1 = pltpu.make_async_remote_copy(
            src_ref=ew_ref.at[hi], dst_ref=slotR.at[hi],
            send_sem=sendA.at[3], recv_sem=recvA.at[3],
            device_id=(left,), device_id_type=pl.DeviceIdType.MESH,
        )
        a_r1.start()
        a_r0.start()
        a_l0.start()
        a_l1.start()

        xv = x_ref[:, :]
        scores = jnp.dot(xv, rw_ref[:, :], preferred_element_type=jnp.float32)
        s_max = jnp.max(scores, axis=-1, keepdims=True)
        p = jnp.exp(scores - s_max)
        p = p / jnp.sum(p, axis=-1, keepdims=True)
        col = lax.broadcasted_iota(jnp.int32, (n_tok, n_exp), 1)
        chosen = jnp.logical_or(col == idx_ref[:, 0:1], col == idx_ref[:, 1:2])
        g = jnp.where(chosen, p, 0.0)
        gates = g / jnp.sum(g, axis=-1, keepdims=True)

        def contrib(src_ref, origin, j0, j1):
            xs_parts = []
            for j in range(j0, j1):
                e_id = origin * e_loc + j
                w = jnp.sum(jnp.where(col == e_id, gates, 0.0), axis=-1)
                xs_parts.append(w[:, None] * xv)
            xs = jnp.concatenate(xs_parts, axis=1)
            W = src_ref[pl.ds(j0, j1 - j0), :, :].reshape((j1 - j0) * d, h)
            return jnp.dot(xs, W, preferred_element_type=jnp.float32)

        out_ref[:, :] = contrib(ew_ref, my, 0, e_loc)

        a_r1.wait_recv()
        b_right = pltpu.make_async_remote_copy(
            src_ref=slotL.at[hi], dst_ref=slotO.at[hi],
            send_sem=sendB.at[0], recv_sem=recvB.at[0],
            device_id=(right,), device_id_type=pl.DeviceIdType.MESH,
        )
        b_right.start()
        a_l0.wait_recv()
        b_left = pltpu.make_async_remote_copy(
            src_ref=slotR.at[lo], dst_ref=slotO.at[lo],
            send_sem=sendB.at[1], recv_sem=recvB.at[1],
            device_id=(left,), device_id_type=pl.DeviceIdType.MESH,
        )
        b_left.start()

        out_ref[:, :] += contrib(slotL, left, e_half, e_loc)
        out_ref[:, :] += contrib(slotR, right, 0, e_half)

        a_r0.wait_recv()
        out_ref[:, :] += contrib(slotL, left, 0, e_half)
        a_l1.wait_recv()
        out_ref[:, :] += contrib(slotR, right, e_half, e_loc)

        opp = lax.rem(my + 2, N_DEV)
        b_right.wait_recv()
        out_ref[:, :] += contrib(slotO, opp, e_half, e_loc)
        b_left.wait_recv()
        out_ref[:, :] += contrib(slotO, opp, 0, e_half)

        for flow in (a_r1, a_r0, a_l0, a_l1, b_right, b_left):
            flow.wait_send()

    return pl.pallas_call(
        body,
        out_shape=jax.ShapeDtypeStruct((n_tok, h), jnp.float32),
        in_specs=[
            pl.BlockSpec(memory_space=pltpu.VMEM),
            pl.BlockSpec(memory_space=pltpu.VMEM),
            pl.BlockSpec(memory_space=pltpu.VMEM),
            pl.BlockSpec(memory_space=pltpu.VMEM),
        ],
        out_specs=pl.BlockSpec(memory_space=pltpu.VMEM),
        scratch_shapes=[
            pltpu.VMEM((e_loc, d, h), jnp.float32),
            pltpu.VMEM((e_loc, d, h), jnp.float32),
            pltpu.VMEM((e_loc, d, h), jnp.float32),
            pltpu.SemaphoreType.DMA((4,)),
            pltpu.SemaphoreType.DMA((4,)),
            pltpu.SemaphoreType.DMA((2,)),
            pltpu.SemaphoreType.DMA((2,)),
        ],
        compiler_params=pltpu.CompilerParams(collective_id=0),
    )(x, router_W, route_idx, expert_W)
